# baseline (speedup 1.0000x reference)
import functools

import jax
import jax.numpy as jnp
from jax import lax
from jax.experimental import pallas as pl
from jax.experimental.pallas import tpu as pltpu
from jax.experimental.pallas import tpu_sc as plsc

_HIDDEN = 1000
_BATCH = 4096
_SEQ = 26
_NC = 2
_NS = 16
_NW = _NC * _NS
_PPW = _BATCH // _NW
_IPW = _PPW * _SEQ
_TAIL = _SEQ - 16


def _build_sc_kernel():
    mesh = plsc.VectorSubcoreMesh(core_axis_name="c", subcore_axis_name="s")

    @functools.partial(
        pl.kernel,
        mesh=mesh,
        compiler_params=pltpu.CompilerParams(needs_layout_passes=False),
        out_type=jax.ShapeDtypeStruct((_BATCH, _SEQ, _HIDDEN), jnp.float32),
        scratch_types=[
            pltpu.VMEM((_SEQ, _HIDDEN), jnp.float32),
            pltpu.VMEM((_SEQ, _HIDDEN), jnp.float32),
            pltpu.VMEM((_IPW + 16,), jnp.int32),
            pltpu.SemaphoreType.DMA,
            pltpu.SemaphoreType.DMA,
        ],
    )
    def onehot(x_hbm, out_hbm, buf0, buf1, idx_v, sem0, sem1):
        cid = lax.axis_index("c")
        sid = lax.axis_index("s")
        wid = sid * _NC + cid
        plane0 = wid * _PPW

        pltpu.sync_copy(x_hbm.at[pl.ds(plane0 * _SEQ, _IPW)],
                        idx_v.at[pl.ds(0, _IPW)])

        lane = lax.iota(jnp.int32, 16)
        zeros = jnp.zeros((16,), jnp.float32)
        ones = jnp.ones((16,), jnp.float32)
        tail8 = lane < 8

        def zero_row(j, carry):
            def zcol(c, carry2):
                buf0[j, pl.ds(c * 16, 16)] = zeros
                buf1[j, pl.ds(c * 16, 16)] = zeros
                return carry2

            lax.fori_loop(0, _HIDDEN // 16, zcol, 0)
            rowv = jnp.full((16,), 1, jnp.int32) * j
            colv = (_HIDDEN // 16) * 16 + lane
            plsc.store_scatter(buf0, [rowv, colv], zeros, mask=tail8)
            plsc.store_scatter(buf1, [rowv, colv], zeros, mask=tail8)
            return carry

        lax.fori_loop(0, _SEQ, zero_row, 0)

        tailmask = lane < _TAIL

        def put(buf, p, val):
            colv0 = idx_v[pl.ds(p * _SEQ, 16)]
            plsc.store_scatter(buf, [lane, colv0], val)
            colv1 = idx_v[pl.ds(p * _SEQ + 16, 16)]
            plsc.store_scatter(buf, [16 + lane, colv1], val, mask=tailmask)

        def start_copy(buf, p, sem):
            return pltpu.async_copy(buf, out_hbm.at[plane0 + p], sem)

        def wait_copy(buf, p, sem):
            pltpu.make_async_copy(buf, out_hbm.at[plane0 + p], sem).wait()

        put(buf0, 0, ones)
        start_copy(buf0, 0, sem0)
        put(buf1, 1, ones)
        start_copy(buf1, 1, sem1)

        def step(i, carry):
            p0 = 2 * i
            wait_copy(buf0, p0 - 2, sem0)
            put(buf0, p0 - 2, zeros)
            put(buf0, p0, ones)
            start_copy(buf0, p0, sem0)

            p1 = p0 + 1
            wait_copy(buf1, p1 - 2, sem1)
            put(buf1, p1 - 2, zeros)
            put(buf1, p1, ones)
            start_copy(buf1, p1, sem1)
            return carry

        lax.fori_loop(1, _PPW // 2, step, 0)

        wait_copy(buf0, _PPW - 2, sem0)
        wait_copy(buf1, _PPW - 1, sem1)

    return onehot


_sc_onehot = _build_sc_kernel()


def kernel(x):
    x_flat = x.reshape(-1).astype(jnp.int32)
    return _sc_onehot(x_flat)
